# baseline (speedup 1.0000x reference)
import functools

import jax
import jax.numpy as jnp
from jax import lax
from jax.experimental import pallas as pl
from jax.experimental.pallas import tpu as pltpu
from jax.experimental.pallas import tpu_sc as plsc

_NC = 2
_NS = 16
_NW = _NC * _NS
_L = 16
_LANES = 128
_SUB = 8


def _sc_hilbert_gather(B, C, H, W, S):
    NR = C * H * W * B // _LANES
    n_btiles = B // _LANES
    n_schunks = S // _LANES
    n_items = C * n_btiles * n_schunks
    per_w = n_items // _NW
    mesh = plsc.VectorSubcoreMesh(core_axis_name="c", subcore_axis_name="s")

    @functools.partial(
        pl.kernel,
        out_type=jax.ShapeDtypeStruct((C, B, S), jnp.float32),
        mesh=mesh,
        scratch_types=[
            pltpu.VMEM((S,), jnp.int32),
            pltpu.VMEM((_LANES,), jnp.int32),
            pltpu.VMEM((_LANES,), jnp.int32),
            pltpu.VMEM((_LANES, _LANES), jnp.float32),
            pltpu.VMEM((_LANES, _LANES), jnp.float32),
            pltpu.VMEM((_LANES, _LANES), jnp.float32),
            pltpu.VMEM((_LANES, _LANES), jnp.float32),
            pltpu.SemaphoreType.DMA,
            pltpu.SemaphoreType.DMA,
            pltpu.SemaphoreType.DMA,
            pltpu.SemaphoreType.DMA,
        ],
        compiler_params=pltpu.CompilerParams(needs_layout_passes=False),
    )
    def k(xr_hbm, idx_hbm, out_hbm, r0_v, ridx0, ridx1, st0, st1, ot0, ot1,
          gsem0, gsem1, osem0, osem1):
        ridxs = (ridx0, ridx1)
        stages = (st0, st1)
        outs = (ot0, ot1)
        gsems = (gsem0, gsem1)
        osems = (osem0, osem1)
        wid = lax.axis_index("s") * _NC + lax.axis_index("c")
        base_item = wid * per_w

        pltpu.sync_copy(idx_hbm, r0_v)

        iota = lax.iota(jnp.int32, _L)

        @plsc.parallel_loop(0, S // _L, unroll=4)
        def _(q):
            p_v = r0_v[pl.ds(q * _L, _L)]
            h_v = p_v // W
            w_v = p_v - h_v * W
            wt_v = w_v // _SUB
            wr_v = w_v - wt_v * _SUB
            r0_v[pl.ds(q * _L, _L)] = (
                h_v * (W * B // _LANES * _SUB)
                + wt_v * (n_btiles * _SUB)
                + wr_v)

        def item_coords(i):
            c = i // (n_btiles * n_schunks)
            rem = i - c * (n_btiles * n_schunks)
            bt = rem // n_schunks
            sc = rem - bt * n_schunks
            return c, bt, sc

        def fill_ridx(i, slot):
            c, bt, sc = item_coords(i)
            off = c * (H * W * B // _LANES) + bt * _SUB
            s0 = sc * _LANES

            @plsc.parallel_loop(0, _LANES // _L, unroll=4)
            def _(q):
                ridxs[slot][pl.ds(q * _L, _L)] = (
                    r0_v[pl.ds(s0 + q * _L, _L)] + off)

        def start_gather(slot):
            pltpu.async_copy(xr_hbm.at[ridxs[slot]], stages[slot], gsems[slot])

        def start_out(i, slot):
            c, bt, sc = item_coords(i)
            pltpu.async_copy(
                outs[slot],
                out_hbm.at[c, pl.ds(bt * _LANES, _LANES),
                           pl.ds(sc * _LANES, _LANES)],
                osems[slot])

        def wait_gather(slot):
            pltpu.make_async_copy(
                xr_hbm.at[ridxs[slot]], stages[slot], gsems[slot]).wait()

        def wait_out(slot):
            pltpu.make_async_copy(
                outs[slot], out_hbm.at[0, pl.ds(0, _LANES), pl.ds(0, _LANES)],
                osems[slot]).wait()

        n_blk = _LANES // _L

        def transpose(slot):
            st = stages[slot]
            ot = outs[slot]

            @plsc.parallel_loop(0, n_blk * n_blk * _L, unroll=8)
            def _(q):
                blk = q // _L
                d = q - blk * _L
                sb = blk // n_blk
                bb = blk - sb * n_blk
                sv0 = sb * _L
                b0 = bb * _L
                rot = (iota + d) & (_L - 1)
                vals = plsc.load_gather(st, [sv0 + iota, b0 + rot])
                plsc.store_scatter(ot, [b0 + rot, sv0 + iota], vals)

        fill_ridx(base_item, 0)
        start_gather(0)

        def body(t, carry):
            for sl in range(2):
                i = base_item + t * 2 + sl
                nsl = 1 - sl
                @pl.when(t * 2 + sl + 1 < per_w)
                def _():
                    fill_ridx(i + 1, nsl)
                    start_gather(nsl)
                wait_gather(sl)
                @pl.when(t * 2 + sl >= 2)
                def _():
                    wait_out(sl)
                transpose(sl)
                start_out(i, sl)
            return carry
        lax.fori_loop(0, per_w // 2, body, 0)

        for sl in range(2):
            wait_out(sl)

    return k


def kernel(x, indices):
    B, C, H, W = x.shape
    S = indices.shape[0]
    NR = C * H * W * B // _LANES
    xr = (x.reshape(B // _LANES, _LANES, C, H, W // _SUB, _SUB)
          .transpose(2, 3, 4, 0, 5, 1)
          .reshape(NR, _LANES))
    idx = indices.astype(jnp.int32)
    yt = _sc_hilbert_gather(B, C, H, W, S)(xr, idx)
    return jnp.transpose(yt, (1, 2, 0))


# speedup vs baseline: 1.0693x; 1.0693x over previous
import functools

import jax
import jax.numpy as jnp
from jax import lax
from jax.experimental import pallas as pl
from jax.experimental.pallas import tpu as pltpu
from jax.experimental.pallas import tpu_sc as plsc

_NC = 2
_NS = 16
_NW = _NC * _NS
_L = 16
_LANES = 128
_SUB = 8


def _sc_hilbert_gather(B, C, H, W, S):
    NR = C * H * W * B // _LANES
    n_btiles = B // _LANES
    n_schunks = S // _LANES
    n_items = C * n_btiles * n_schunks
    per_w = n_items // _NW
    mesh = plsc.VectorSubcoreMesh(core_axis_name="c", subcore_axis_name="s")

    @functools.partial(
        pl.kernel,
        out_type=jax.ShapeDtypeStruct((C, B, S), jnp.float32),
        mesh=mesh,
        scratch_types=[
            pltpu.VMEM((S,), jnp.int32),
            pltpu.VMEM((_LANES,), jnp.int32),
            pltpu.VMEM((_LANES,), jnp.int32),
            pltpu.VMEM((_LANES,), jnp.int32),
            pltpu.VMEM((_LANES, _LANES), jnp.float32),
            pltpu.VMEM((_LANES, _LANES), jnp.float32),
            pltpu.VMEM((_LANES, _LANES), jnp.float32),
            pltpu.VMEM((_LANES, _LANES), jnp.float32),
            pltpu.VMEM((_LANES, _LANES), jnp.float32),
            pltpu.VMEM((_LANES, _LANES), jnp.float32),
            pltpu.SemaphoreType.DMA,
            pltpu.SemaphoreType.DMA,
            pltpu.SemaphoreType.DMA,
            pltpu.SemaphoreType.DMA,
            pltpu.SemaphoreType.DMA,
            pltpu.SemaphoreType.DMA,
        ],
        compiler_params=pltpu.CompilerParams(needs_layout_passes=False),
    )
    def k(xr_hbm, idx_hbm, out_hbm, r0_v, ridx0, ridx1, ridx2,
          st0, st1, st2, ot0, ot1, ot2,
          gsem0, gsem1, gsem2, osem0, osem1, osem2):
        ridxs = (ridx0, ridx1, ridx2)
        stages = (st0, st1, st2)
        outs = (ot0, ot1, ot2)
        gsems = (gsem0, gsem1, gsem2)
        osems = (osem0, osem1, osem2)
        wid = lax.axis_index("s") * _NC + lax.axis_index("c")
        base_item = wid * per_w

        pltpu.sync_copy(idx_hbm, r0_v)

        iota = lax.iota(jnp.int32, _L)

        @plsc.parallel_loop(0, S // _L, unroll=4)
        def _(q):
            p_v = r0_v[pl.ds(q * _L, _L)]
            h_v = p_v // W
            w_v = p_v - h_v * W
            wt_v = w_v // _SUB
            wr_v = w_v - wt_v * _SUB
            r0_v[pl.ds(q * _L, _L)] = (
                h_v * (W * B // _LANES * _SUB)
                + wt_v * (n_btiles * _SUB)
                + wr_v)

        def item_coords(i):
            c = i // (n_btiles * n_schunks)
            rem = i - c * (n_btiles * n_schunks)
            bt = rem // n_schunks
            sc = rem - bt * n_schunks
            return c, bt, sc

        def fill_ridx(i, slot):
            c, bt, sc = item_coords(i)
            off = c * (H * W * B // _LANES) + bt * _SUB
            s0 = sc * _LANES

            @plsc.parallel_loop(0, _LANES // _L, unroll=4)
            def _(q):
                ridxs[slot][pl.ds(q * _L, _L)] = (
                    r0_v[pl.ds(s0 + q * _L, _L)] + off)

        def start_gather(slot):
            pltpu.async_copy(xr_hbm.at[ridxs[slot]], stages[slot], gsems[slot])

        def start_out(i, slot):
            c, bt, sc = item_coords(i)
            pltpu.async_copy(
                outs[slot],
                out_hbm.at[c, pl.ds(bt * _LANES, _LANES),
                           pl.ds(sc * _LANES, _LANES)],
                osems[slot])

        def wait_gather(slot):
            pltpu.make_async_copy(
                xr_hbm.at[ridxs[slot]], stages[slot], gsems[slot]).wait()

        def wait_out(slot):
            pltpu.make_async_copy(
                outs[slot], out_hbm.at[0, pl.ds(0, _LANES), pl.ds(0, _LANES)],
                osems[slot]).wait()

        n_blk = _LANES // _L

        def transpose(slot):
            st = stages[slot]
            ot = outs[slot]

            @plsc.parallel_loop(0, n_blk * n_blk * _L, unroll=8)
            def _(q):
                blk = q // _L
                d = q - blk * _L
                sb = blk // n_blk
                bb = blk - sb * n_blk
                sv0 = sb * _L
                b0 = bb * _L
                rot = (iota + d) & (_L - 1)
                vals = plsc.load_gather(st, [sv0 + iota, b0 + rot])
                plsc.store_scatter(ot, [b0 + rot, sv0 + iota], vals)

        fill_ridx(base_item, 0)
        start_gather(0)
        fill_ridx(base_item + 1, 1)
        start_gather(1)

        def body(t, carry):
            for sl in range(3):
                i = base_item + t * 3 + sl
                nsl = (sl + 2) % 3
                @pl.when(t * 3 + sl + 2 < per_w)
                def _():
                    fill_ridx(i + 2, nsl)
                    start_gather(nsl)
                wait_gather(sl)
                @pl.when(t * 3 + sl >= 3)
                def _():
                    wait_out(sl)
                transpose(sl)
                start_out(i, sl)
            return carry
        lax.fori_loop(0, per_w // 3, body, 0)

        for sl in range(3):
            wait_out(sl)

    return k


def kernel(x, indices):
    B, C, H, W = x.shape
    S = indices.shape[0]
    NR = C * H * W * B // _LANES
    xr = (x.reshape(B // _LANES, _LANES, C, H, W // _SUB, _SUB)
          .transpose(2, 3, 4, 0, 5, 1)
          .reshape(NR, _LANES))
    idx = indices.astype(jnp.int32)
    yt = _sc_hilbert_gather(B, C, H, W, S)(xr, idx)
    return jnp.transpose(yt, (1, 2, 0))
